# Initial kernel scaffold; baseline (speedup 1.0000x reference)
import jax
import jax.numpy as jnp
from jax.experimental import pallas as pl


def kernel(user_w, item_w, edge_vals, user, pos, neg, edge_rows, edge_cols):
    raise NotImplementedError("write your pallas kernel here")


# speedup vs baseline: 6.2737x; 6.2737x over previous
import functools

import jax
import jax.numpy as jnp
from jax import lax
from jax.experimental import pallas as pl
from jax.experimental.pallas import tpu as pltpu
from jax.experimental.pallas import tpu_sc as plsc

N_USER = 50000
N_ITEM = 50000
EMBED = 64
HALF = 32
NUM_GC = 3
WEIGHT_DECAY = 1e-4
BATCH = 4096
NUM_EDGES = 800000

NC = 2
NS = 16
LANES = 16

CLEN = 128
CHUNKS = 392
E_PAD = NS * CHUNKS * CLEN
GROUP = 56
N_GROUPS = CHUNKS // GROUP

N_PAD = 50176
ZROWS = 56
FLUSH_ROWS = N_PAD // NS

_MESH = plsc.VectorSubcoreMesh(
    core_axis_name="c", subcore_axis_name="s", num_cores=NC, num_subcores=NS)
_SC_PARAMS = pltpu.CompilerParams(use_tc_tiling_on_sc=False)


def _scale_rows(rows_ref, val_ref, j):
  def body(eb, _):
    vv = val_ref[j, pl.ds(eb * LANES, LANES)]
    for l in range(LANES):
      e = eb * LANES + l
      v = vv[l]
      rows_ref[e, pl.ds(0, LANES)] = rows_ref[e, pl.ds(0, LANES)] * v
      rows_ref[e, pl.ds(LANES, LANES)] = rows_ref[e, pl.ds(LANES, LANES)] * v
    return 0
  lax.fori_loop(0, CLEN // LANES, body, 0)


def _propagate_body(tlo, thi, dst_hbm, src_hbm, val_hbm, out_lo, out_hi,
                    acc, dstb, srcb, valb, rows, zbuf, sem):
  c = lax.axis_index("c")
  s = lax.axis_index("s")

  zero = jnp.zeros((LANES,), jnp.float32)
  def zrow(i, _):
    zbuf[i, pl.ds(0, LANES)] = zero
    zbuf[i, pl.ds(LANES, LANES)] = zero
    return 0
  lax.fori_loop(0, ZROWS, zrow, 0)
  base = s * (N_PAD // NS)
  def zcopy(r, _):
    pltpu.sync_copy(zbuf, acc.at[pl.ds(base + r * ZROWS, ZROWS)])
    return 0
  lax.fori_loop(0, (N_PAD // NS) // ZROWS, zcopy, 0)
  plsc.subcore_barrier()

  for g in range(N_GROUPS):
    pltpu.sync_copy(dst_hbm.at[s, pl.ds(g * GROUP, GROUP)], dstb)
    pltpu.sync_copy(src_hbm.at[s, pl.ds(g * GROUP, GROUP)], srcb)
    pltpu.sync_copy(val_hbm.at[s, pl.ds(g * GROUP, GROUP)], valb)

    def chunk(j, _):
      @pl.when(c == 0)
      def _():
        pltpu.async_copy(tlo.at[srcb.at[j]], rows, sem).wait()
      @pl.when(c == 1)
      def _():
        pltpu.async_copy(thi.at[srcb.at[j]], rows, sem).wait()
      _scale_rows(rows, valb, j)
      pltpu.sync_copy(rows, acc.at[dstb.at[j]], add=True)
      return 0
    lax.fori_loop(0, GROUP, chunk, 0)

  plsc.subcore_barrier()

  fbase = s * FLUSH_ROWS
  @pl.when(c == 0)
  def _():
    pltpu.sync_copy(acc.at[pl.ds(fbase, FLUSH_ROWS)],
                    out_lo.at[pl.ds(fbase, FLUSH_ROWS)])
  @pl.when(c == 1)
  def _():
    pltpu.sync_copy(acc.at[pl.ds(fbase, FLUSH_ROWS)],
                    out_hi.at[pl.ds(fbase, FLUSH_ROWS)])


_propagate = functools.partial(
    pl.kernel,
    out_type=[jax.ShapeDtypeStruct((N_PAD, HALF), jnp.float32),
              jax.ShapeDtypeStruct((N_PAD, HALF), jnp.float32)],
    mesh=_MESH,
    scratch_types=[
        pltpu.VMEM_SHARED((N_PAD, HALF), jnp.float32),
        pltpu.VMEM((GROUP, CLEN), jnp.int32),
        pltpu.VMEM((GROUP, CLEN), jnp.int32),
        pltpu.VMEM((GROUP, CLEN), jnp.float32),
        pltpu.VMEM((CLEN, HALF), jnp.float32),
        pltpu.VMEM((ZROWS, HALF), jnp.float32),
        pltpu.SemaphoreType.DMA,
    ],
    compiler_params=_SC_PARAMS,
)(_propagate_body)


def _gather_body(ulo, uhi, ilo, ihi, idx_u, idx_i, out_lo, out_hi,
                 iub, iib, rows, sem):
  c = lax.axis_index("c")
  s = lax.axis_index("s")
  pltpu.sync_copy(idx_u.at[s], iub)
  pltpu.sync_copy(idx_i.at[s], iib)

  def emit(table, out):
    for k in range(2):
      pltpu.async_copy(table[0].at[iub.at[k]], rows, sem).wait()
      pltpu.sync_copy(rows, out.at[s, pl.ds(k * CLEN, CLEN)])
    for k in range(4):
      pltpu.async_copy(table[1].at[iib.at[k]], rows, sem).wait()
      pltpu.sync_copy(rows, out.at[s, pl.ds((2 + k) * CLEN, CLEN)])

  @pl.when(c == 0)
  def _():
    emit((ulo, ilo), out_lo)
  @pl.when(c == 1)
  def _():
    emit((uhi, ihi), out_hi)


_gather = functools.partial(
    pl.kernel,
    out_type=[jax.ShapeDtypeStruct((NS, 6 * CLEN, HALF), jnp.float32),
              jax.ShapeDtypeStruct((NS, 6 * CLEN, HALF), jnp.float32)],
    mesh=_MESH,
    scratch_types=[
        pltpu.VMEM((2, CLEN), jnp.int32),
        pltpu.VMEM((4, CLEN), jnp.int32),
        pltpu.VMEM((CLEN, HALF), jnp.float32),
        pltpu.SemaphoreType.DMA,
    ],
    compiler_params=_SC_PARAMS,
)(_gather_body)


def _loss_body(u_ref, p_ref, n_ref, out_ref):
  u = (u_ref[0] + u_ref[1] + u_ref[2] + u_ref[3]) * 0.25
  p = (p_ref[0] + p_ref[1] + p_ref[2] + p_ref[3]) * 0.25
  n = (n_ref[0] + n_ref[1] + n_ref[2] + n_ref[3]) * 0.25
  pos_out = jnp.sum(u * p, axis=1)
  neg_out = jnp.sum(u * n, axis=1)
  out = pos_out - neg_out
  loss = jnp.sum(jax.nn.log_sigmoid(out))
  reg = WEIGHT_DECAY * 0.5 * (
      jnp.sum(u_ref[0] * u_ref[0]) + jnp.sum(p_ref[0] * p_ref[0])
      + jnp.sum(n_ref[0] * n_ref[0])) / float(N_USER)
  out_ref[0, 0] = -loss + reg


def _loss_call(u_stack, p_stack, n_stack):
  return pl.pallas_call(
      _loss_body,
      out_shape=jax.ShapeDtypeStruct((1, 1), jnp.float32),
      in_specs=[pl.BlockSpec(memory_space=pltpu.VMEM)] * 3,
      out_specs=pl.BlockSpec(memory_space=pltpu.SMEM),
  )(u_stack, p_stack, n_stack)


def _split(table):
  t = table.reshape(-1, 2, HALF)
  t = jnp.pad(t, ((0, N_PAD - t.shape[0]), (0, 0), (0, 0)))
  return t[:, 0, :], t[:, 1, :]


def kernel(user_w, item_w, edge_vals, user, pos, neg, edge_rows, edge_cols):
  i32 = jnp.int32
  pad = E_PAD - NUM_EDGES
  rows_p = jnp.pad(edge_rows.astype(i32), (0, pad)).reshape(NS, CHUNKS, CLEN)
  cols_p = jnp.pad(edge_cols.astype(i32), (0, pad)).reshape(NS, CHUNKS, CLEN)
  vals_p = jnp.pad(edge_vals, (0, pad)).reshape(NS, CHUNKS, CLEN)

  idx_u = user.astype(i32).reshape(NS, 2, CLEN)
  idx_i = jnp.concatenate(
      [pos.astype(i32).reshape(NS, 2, CLEN),
       neg.astype(i32).reshape(NS, 2, CLEN)], axis=1)

  ulo, uhi = _split(user_w)
  ilo, ihi = _split(item_w)

  gathers = [_gather(ulo, uhi, ilo, ihi, idx_u, idx_i)]
  cu, ci = (ulo, uhi), (ilo, ihi)
  for _ in range(NUM_GC):
    cu = _propagate(ci[0], ci[1], rows_p, cols_p, vals_p)
    ci = _propagate(cu[0], cu[1], cols_p, rows_p, vals_p)
    gathers.append(_gather(cu[0], cu[1], ci[0], ci[1], idx_u, idx_i))

  def assemble(slabs):
    full = jnp.stack(slabs, axis=2)
    full = full.reshape(NS, 6 * CLEN, EMBED)
    u = full[:, :2 * CLEN].reshape(BATCH, EMBED)
    p = full[:, 2 * CLEN:4 * CLEN].reshape(BATCH, EMBED)
    n = full[:, 4 * CLEN:].reshape(BATCH, EMBED)
    return u, p, n

  us, ps, ns_ = zip(*(assemble(g) for g in gathers))
  loss = _loss_call(jnp.stack(us), jnp.stack(ps), jnp.stack(ns_))
  return loss[0, 0]


# speedup vs baseline: 8.2632x; 1.3171x over previous
import functools

import jax
import jax.numpy as jnp
from jax import lax
from jax.experimental import pallas as pl
from jax.experimental.pallas import tpu as pltpu
from jax.experimental.pallas import tpu_sc as plsc

N_USER = 50000
N_ITEM = 50000
EMBED = 64
HALF = 32
NUM_GC = 3
WEIGHT_DECAY = 1e-4
BATCH = 4096
NUM_EDGES = 800000

NC = 2
NS = 16
LANES = 16

CLEN = 128
CHUNKS = 392
E_PAD = NS * CHUNKS * CLEN
GROUP = 28
N_GROUPS = CHUNKS // GROUP
NPAIR = GROUP // 2

N_PAD = 50176
ZROWS = 56
FLUSH_ROWS = N_PAD // NS

_MESH = plsc.VectorSubcoreMesh(
    core_axis_name="c", subcore_axis_name="s", num_cores=NC, num_subcores=NS)
_SC_PARAMS = pltpu.CompilerParams(use_tc_tiling_on_sc=False)


def _scale_rows(rows_ref, val_ref, j):
  def body(eb, _):
    vv = val_ref[j, pl.ds(eb * LANES, LANES)]
    for l in range(LANES):
      e = eb * LANES + l
      v = vv[l]
      rows_ref[e, pl.ds(0, LANES)] = rows_ref[e, pl.ds(0, LANES)] * v
      rows_ref[e, pl.ds(LANES, LANES)] = rows_ref[e, pl.ds(LANES, LANES)] * v
    return 0
  lax.fori_loop(0, CLEN // LANES, body, 0)


def _propagate_body(tlo, thi, dst_hbm, src_hbm, val_hbm, out_lo, out_hi,
                    acc, dstb, srcb, valb, rows0, rows1, zbuf, sem):
  c = lax.axis_index("c")
  s = lax.axis_index("s")

  zero = jnp.zeros((LANES,), jnp.float32)
  def zrow(i, _):
    zbuf[i, pl.ds(0, LANES)] = zero
    zbuf[i, pl.ds(LANES, LANES)] = zero
    return 0
  lax.fori_loop(0, ZROWS, zrow, 0)
  base = s * (N_PAD // NS)
  def zcopy(r, _):
    pltpu.sync_copy(zbuf, acc.at[pl.ds(base + r * ZROWS, ZROWS)])
    return 0
  lax.fori_loop(0, (N_PAD // NS) // ZROWS, zcopy, 0)
  plsc.subcore_barrier()

  def fire(j, buf):
    @pl.when(c == 0)
    def _():
      pltpu.async_copy(tlo.at[srcb.at[j]], buf, sem)
    @pl.when(c == 1)
    def _():
      pltpu.async_copy(thi.at[srcb.at[j]], buf, sem)

  def wait_gather(buf):
    pltpu.make_async_copy(tlo.at[srcb.at[0]], buf, sem).wait()

  for g in range(N_GROUPS):
    pltpu.sync_copy(dst_hbm.at[s, pl.ds(g * GROUP, GROUP)], dstb)
    pltpu.sync_copy(src_hbm.at[s, pl.ds(g * GROUP, GROUP)], srcb)
    pltpu.sync_copy(val_hbm.at[s, pl.ds(g * GROUP, GROUP)], valb)

    fire(0, rows0)

    def pair(jj, _):
      j0 = 2 * jj
      j1 = j0 + 1
      wait_gather(rows0)
      fire(j1, rows1)
      _scale_rows(rows0, valb, j0)
      pltpu.sync_copy(rows0, acc.at[dstb.at[j0]], add=True)
      wait_gather(rows1)
      @pl.when(jj < NPAIR - 1)
      def _():
        fire(j1 + 1, rows0)
      _scale_rows(rows1, valb, j1)
      pltpu.sync_copy(rows1, acc.at[dstb.at[j1]], add=True)
      return 0
    lax.fori_loop(0, NPAIR, pair, 0)

  plsc.subcore_barrier()

  fbase = s * FLUSH_ROWS
  @pl.when(c == 0)
  def _():
    pltpu.sync_copy(acc.at[pl.ds(fbase, FLUSH_ROWS)],
                    out_lo.at[pl.ds(fbase, FLUSH_ROWS)])
  @pl.when(c == 1)
  def _():
    pltpu.sync_copy(acc.at[pl.ds(fbase, FLUSH_ROWS)],
                    out_hi.at[pl.ds(fbase, FLUSH_ROWS)])


_propagate = functools.partial(
    pl.kernel,
    out_type=[jax.ShapeDtypeStruct((N_PAD, HALF), jnp.float32),
              jax.ShapeDtypeStruct((N_PAD, HALF), jnp.float32)],
    mesh=_MESH,
    scratch_types=[
        pltpu.VMEM_SHARED((N_PAD, HALF), jnp.float32),
        pltpu.VMEM((GROUP, CLEN), jnp.int32),
        pltpu.VMEM((GROUP, CLEN), jnp.int32),
        pltpu.VMEM((GROUP, CLEN), jnp.float32),
        pltpu.VMEM((CLEN, HALF), jnp.float32),
        pltpu.VMEM((CLEN, HALF), jnp.float32),
        pltpu.VMEM((ZROWS, HALF), jnp.float32),
        pltpu.SemaphoreType.DMA,
    ],
    compiler_params=_SC_PARAMS,
)(_propagate_body)


def _gather_body(ulo, uhi, ilo, ihi, idx_u, idx_i, out_lo, out_hi,
                 iub, iib, rows, sem):
  c = lax.axis_index("c")
  s = lax.axis_index("s")
  pltpu.sync_copy(idx_u.at[s], iub)
  pltpu.sync_copy(idx_i.at[s], iib)

  def emit(table, out):
    for k in range(2):
      pltpu.async_copy(table[0].at[iub.at[k]], rows, sem).wait()
      pltpu.sync_copy(rows, out.at[s, pl.ds(k * CLEN, CLEN)])
    for k in range(4):
      pltpu.async_copy(table[1].at[iib.at[k]], rows, sem).wait()
      pltpu.sync_copy(rows, out.at[s, pl.ds((2 + k) * CLEN, CLEN)])

  @pl.when(c == 0)
  def _():
    emit((ulo, ilo), out_lo)
  @pl.when(c == 1)
  def _():
    emit((uhi, ihi), out_hi)


_gather = functools.partial(
    pl.kernel,
    out_type=[jax.ShapeDtypeStruct((NS, 6 * CLEN, HALF), jnp.float32),
              jax.ShapeDtypeStruct((NS, 6 * CLEN, HALF), jnp.float32)],
    mesh=_MESH,
    scratch_types=[
        pltpu.VMEM((2, CLEN), jnp.int32),
        pltpu.VMEM((4, CLEN), jnp.int32),
        pltpu.VMEM((CLEN, HALF), jnp.float32),
        pltpu.SemaphoreType.DMA,
    ],
    compiler_params=_SC_PARAMS,
)(_gather_body)


def _loss_body(u_ref, p_ref, n_ref, out_ref):
  u = (u_ref[0] + u_ref[1] + u_ref[2] + u_ref[3]) * 0.25
  p = (p_ref[0] + p_ref[1] + p_ref[2] + p_ref[3]) * 0.25
  n = (n_ref[0] + n_ref[1] + n_ref[2] + n_ref[3]) * 0.25
  pos_out = jnp.sum(u * p, axis=1)
  neg_out = jnp.sum(u * n, axis=1)
  out = pos_out - neg_out
  loss = jnp.sum(jax.nn.log_sigmoid(out))
  reg = WEIGHT_DECAY * 0.5 * (
      jnp.sum(u_ref[0] * u_ref[0]) + jnp.sum(p_ref[0] * p_ref[0])
      + jnp.sum(n_ref[0] * n_ref[0])) / float(N_USER)
  out_ref[0, 0] = -loss + reg


def _loss_call(u_stack, p_stack, n_stack):
  return pl.pallas_call(
      _loss_body,
      out_shape=jax.ShapeDtypeStruct((1, 1), jnp.float32),
      in_specs=[pl.BlockSpec(memory_space=pltpu.VMEM)] * 3,
      out_specs=pl.BlockSpec(memory_space=pltpu.SMEM),
  )(u_stack, p_stack, n_stack)


def _split(table):
  t = table.reshape(-1, 2, HALF)
  t = jnp.pad(t, ((0, N_PAD - t.shape[0]), (0, 0), (0, 0)))
  return t[:, 0, :], t[:, 1, :]


def kernel(user_w, item_w, edge_vals, user, pos, neg, edge_rows, edge_cols):
  i32 = jnp.int32
  pad = E_PAD - NUM_EDGES
  rows_p = jnp.pad(edge_rows.astype(i32), (0, pad)).reshape(NS, CHUNKS, CLEN)
  cols_p = jnp.pad(edge_cols.astype(i32), (0, pad)).reshape(NS, CHUNKS, CLEN)
  vals_p = jnp.pad(edge_vals, (0, pad)).reshape(NS, CHUNKS, CLEN)

  idx_u = user.astype(i32).reshape(NS, 2, CLEN)
  idx_i = jnp.concatenate(
      [pos.astype(i32).reshape(NS, 2, CLEN),
       neg.astype(i32).reshape(NS, 2, CLEN)], axis=1)

  ulo, uhi = _split(user_w)
  ilo, ihi = _split(item_w)

  gathers = [_gather(ulo, uhi, ilo, ihi, idx_u, idx_i)]
  cu, ci = (ulo, uhi), (ilo, ihi)
  for _ in range(NUM_GC):
    cu = _propagate(ci[0], ci[1], rows_p, cols_p, vals_p)
    ci = _propagate(cu[0], cu[1], cols_p, rows_p, vals_p)
    gathers.append(_gather(cu[0], cu[1], ci[0], ci[1], idx_u, idx_i))

  def assemble(slabs):
    full = jnp.stack(slabs, axis=2)
    full = full.reshape(NS, 6 * CLEN, EMBED)
    u = full[:, :2 * CLEN].reshape(BATCH, EMBED)
    p = full[:, 2 * CLEN:4 * CLEN].reshape(BATCH, EMBED)
    n = full[:, 4 * CLEN:].reshape(BATCH, EMBED)
    return u, p, n

  us, ps, ns_ = zip(*(assemble(g) for g in gathers))
  loss = _loss_call(jnp.stack(us), jnp.stack(ps), jnp.stack(ns_))
  return loss[0, 0]
